# baseline (speedup 1.0000x reference)
import jax
import jax.numpy as jnp
from jax import lax
from jax.experimental import pallas as pl

IMG_W = 512.0
IMG_H = 512.0


def _body(bb_ref, inp_ref, wt_ref, b_ref, out_ref):
    fm_w = 32.0
    fm_h = 32.0
    cell_w = IMG_W / fm_w
    cell_h = IMG_H / fm_h

    bb = bb_ref[0]
    x = bb[:, 0:1]
    y = bb[:, 1:2]
    w = bb[:, 2:3]
    h = bb[:, 3:4]
    xc = x + jnp.floor(w / 2.0)
    yc = y + jnp.floor(h / 2.0)
    cxf = jnp.floor(xc / cell_w)
    cyf = jnp.floor(yc / cell_h)
    off = cyf * fm_w + cxf

    iot = lax.broadcasted_iota(jnp.int32, (8, 1024), 1)
    onehot = (iot == off.astype(jnp.int32)).astype(jnp.float32)
    feat = lax.dot_general(
        onehot, inp_ref[0],
        dimension_numbers=(((1,), (1,)), ((), ())),
        preferred_element_type=jnp.float32,
    )

    lab = jnp.concatenate(
        [(xc - cxf * cell_w) / cell_w,
         (yc - cyf * cell_h) / cell_h,
         w / IMG_W,
         h / IMG_H], axis=1)
    lin = jnp.dot(lab, wt_ref[...],
                  preferred_element_type=jnp.float32)
    out_ref[0] = feat + lin + b_ref[...]


def kernel(input, bboxes, W, b):
    B, K, N, _ = bboxes.shape
    BK = B * K
    C = input.shape[1]
    P = input.shape[2] * input.shape[3]
    inp_r = input.reshape(BK, C, P)
    bb_r = bboxes.reshape(BK, N, 4)
    wt = W.T
    b2 = b.reshape(1, C)

    out = pl.pallas_call(
        _body,
        grid=(BK,),
        in_specs=[
            pl.BlockSpec((1, N, 4), lambda i: (i, 0, 0)),
            pl.BlockSpec((1, C, P), lambda i: (i, 0, 0)),
            pl.BlockSpec((4, C), lambda i: (0, 0)),
            pl.BlockSpec((1, C), lambda i: (0, 0)),
        ],
        out_specs=pl.BlockSpec((1, N, C), lambda i: (i, 0, 0)),
        out_shape=jax.ShapeDtypeStruct((BK, N, C), jnp.float32),
    )(bb_r, inp_r, wt, b2)
    return out.reshape(B, K, N, C)
